# baseline (speedup 1.0000x reference)
import functools

import jax
import jax.numpy as jnp
from jax import lax
from jax.experimental import pallas as pl
from jax.experimental.pallas import tpu as pltpu
from jax.experimental.pallas import tpu_sc as plsc

_WB = 256


def _transpose_pad(tblT, tail128, *, num_cores, num_subcores):
    D, V = tblT.shape
    nw = num_cores * num_subcores
    nb = V // _WB
    tail = V - nb * _WB
    nk = (nb + nw - 1) // nw
    nk += nk % 2

    mesh = plsc.VectorSubcoreMesh(core_axis_name="c", subcore_axis_name="s")

    @functools.partial(
        pl.kernel,
        out_type=jax.ShapeDtypeStruct((V, 128), jnp.float32),
        mesh=mesh,
        compiler_params=pltpu.CompilerParams(
            use_tc_tiling_on_sc=True, skip_device_barrier=True,
            needs_layout_passes=False),
        scratch_types=[
            pltpu.VMEM((D, _WB), jnp.float32),
            pltpu.VMEM((D, _WB), jnp.float32),
            pltpu.VMEM((_WB, 128), jnp.float32),
            pltpu.VMEM((_WB, 128), jnp.float32),
            pltpu.SemaphoreType.DMA,
            pltpu.SemaphoreType.DMA,
            pltpu.SemaphoreType.DMA,
            pltpu.SemaphoreType.DMA,
        ],
    )
    def tp(tblT_hbm, tail_hbm, out_hbm, in0, in1, ou0, ou1,
           isem0, isem1, osem0, osem1):
        wid = lax.axis_index("s") * num_cores + lax.axis_index("c")
        ins = (in0, in1)
        ous = (ou0, ou1)
        isems = (isem0, isem1)
        osems = (osem0, osem1)

        def blk(ki):
            k = jnp.minimum(wid + ki * nw, nb - 1)
            return pl.multiple_of(k * _WB, _WB)

        def start_in(ki, b):
            pltpu.async_copy(
                tblT_hbm.at[:, pl.ds(blk(ki), _WB)], ins[b], isems[b])

        def wait_in(b):
            pltpu.make_async_copy(
                tblT_hbm.at[:, pl.ds(0, _WB)], ins[b], isems[b]).wait()

        def start_out(ki, b):
            pltpu.async_copy(
                ous[b], out_hbm.at[pl.ds(blk(ki), _WB)], osems[b])

        def wait_out(b):
            pltpu.make_async_copy(
                ous[b], out_hbm.at[pl.ds(0, _WB)], osems[b]).wait()

        @pl.when(wid == nw - 1)
        def _tail():
            pltpu.sync_copy(tail_hbm, out_hbm.at[pl.ds(nb * _WB, tail)])

        start_in(0, 0)
        start_in(1, 1)
        lanes = jax.lax.iota(jnp.int32, 16)

        @pl.loop(0, nk // 2)
        def _body(o):
            for b in range(2):
                ki = o * 2 + b
                wait_in(b)

                @pl.when(ki >= 2)
                def _drain():
                    wait_out(b)

                @pl.loop(0, _WB // 16)
                def _rows(r0):
                    r16 = pl.multiple_of(r0 * 16, 16)
                    row_ids = r16 + lanes
                    for d in range(D):
                        v = ins[b][d, pl.ds(r16, 16)]
                        plsc.store_scatter(
                            ous[b],
                            [row_ids, jnp.full((16,), d, jnp.int32)],
                            v)

                start_out(ki, b)

                @pl.when(ki + 2 < nk)
                def _refill():
                    start_in(ki + 2, b)

        wait_out(0)
        wait_out(1)

    return tp(tblT, tail128)


def _emb_lookup(idx, table, *, num_cores, num_subcores, chunk):
    B, = idx.shape
    V, D = table.shape
    nw = num_cores * num_subcores
    b_per_w = B // nw
    nchunks = b_per_w // chunk

    mesh = plsc.VectorSubcoreMesh(core_axis_name="c", subcore_axis_name="s")

    @functools.partial(
        pl.kernel,
        out_type=jax.ShapeDtypeStruct((B, D), jnp.float32),
        mesh=mesh,
        compiler_params=pltpu.CompilerParams(
            use_tc_tiling_on_sc=True, skip_device_barrier=True),
        scratch_types=[
            pltpu.VMEM((b_per_w,), jnp.int32),
            pltpu.VMEM((chunk, D), jnp.float32),
            pltpu.VMEM((chunk, D), jnp.float32),
            pltpu.SemaphoreType.DMA,
            pltpu.SemaphoreType.DMA,
            pltpu.SemaphoreType.DMA,
            pltpu.SemaphoreType.DMA,
        ],
    )
    def emb(idx_hbm, table_hbm, out_hbm, idx_v, rows0, rows1,
            gsem0, gsem1, ssem0, ssem1):
        wid = lax.axis_index("s") * num_cores + lax.axis_index("c")
        base = wid * b_per_w
        pltpu.sync_copy(idx_hbm.at[pl.ds(base, b_per_w)], idx_v)

        bufs = (rows0, rows1)
        gsems = (gsem0, gsem1)
        ssems = (ssem0, ssem1)

        def start_gather(g, b):
            pltpu.async_copy(
                table_hbm.at[idx_v.at[pl.ds(g * chunk, chunk)]],
                bufs[b], gsems[b])

        def start_store(g, b):
            pltpu.async_copy(
                bufs[b], out_hbm.at[pl.ds(base + g * chunk, chunk)],
                ssems[b])

        def wait_gather(b):
            pltpu.make_async_copy(
                table_hbm.at[pl.ds(0, chunk)], bufs[b], gsems[b]).wait()

        def wait_store(b):
            pltpu.make_async_copy(
                bufs[b], out_hbm.at[pl.ds(base, chunk)], ssems[b]).wait()

        start_gather(0, 0)
        start_gather(1, 1)

        @pl.loop(0, nchunks // 2)
        def _body(o):
            for b in range(2):
                g = o * 2 + b
                wait_gather(b)
                start_store(g, b)

                @pl.when(g + 2 < nchunks)
                def _refill():
                    wait_store(b)
                    start_gather(g + 2, b)

        wait_store(0)
        wait_store(1)

    return emb(idx, table)


def kernel(X, table):
    B0, B1 = X.shape
    B = B0 * B1
    V, D = table.shape
    idx = X.reshape(B).astype(jnp.int32)
    info = plsc.get_sparse_core_info()
    nw = info.num_cores * info.num_subcores

    nb = V // _WB
    tail128 = jnp.pad(table[nb * _WB:, :], ((0, 0), (0, 128 - D)))
    tbl128 = _transpose_pad(
        table.T, tail128,
        num_cores=info.num_cores, num_subcores=info.num_subcores)

    out = _emb_lookup(
        idx,
        tbl128,
        num_cores=info.num_cores,
        num_subcores=info.num_subcores,
        chunk=256,
    )
    return out.reshape(B0, B1, 128)[:, :, :D]


# speedup vs baseline: 1.4958x; 1.4958x over previous
import functools

import jax
import jax.numpy as jnp
from jax import lax
from jax.experimental import pallas as pl
from jax.experimental.pallas import tpu as pltpu
from jax.experimental.pallas import tpu_sc as plsc

_WB = 256


def _transpose_pad(tblT, tail128, *, num_cores, num_subcores):
    D, V = tblT.shape
    nw = num_cores * num_subcores
    nb = V // _WB
    tail = V - nb * _WB
    nk = (nb + nw - 1) // nw
    nk += nk % 2

    mesh = plsc.VectorSubcoreMesh(core_axis_name="c", subcore_axis_name="s")

    @functools.partial(
        pl.kernel,
        out_type=jax.ShapeDtypeStruct((V, 128), jnp.float32),
        mesh=mesh,
        compiler_params=pltpu.CompilerParams(
            use_tc_tiling_on_sc=True, skip_device_barrier=True,
            needs_layout_passes=False),
        scratch_types=[
            pltpu.VMEM((D, _WB), jnp.float32),
            pltpu.VMEM((D, _WB), jnp.float32),
            pltpu.VMEM((_WB, 128), jnp.float32),
            pltpu.VMEM((_WB, 128), jnp.float32),
            pltpu.SemaphoreType.DMA,
            pltpu.SemaphoreType.DMA,
            pltpu.SemaphoreType.DMA,
            pltpu.SemaphoreType.DMA,
        ],
    )
    def tp(tblT_hbm, tail_hbm, out_hbm, in0, in1, ou0, ou1,
           isem0, isem1, osem0, osem1):
        wid = lax.axis_index("s") * num_cores + lax.axis_index("c")
        ins = (in0, in1)
        ous = (ou0, ou1)
        isems = (isem0, isem1)
        osems = (osem0, osem1)

        def blk(ki):
            k = jnp.minimum(wid + ki * nw, nb - 1)
            return pl.multiple_of(k * _WB, _WB)

        def start_in(ki, b):
            pltpu.async_copy(
                tblT_hbm.at[:, pl.ds(blk(ki), _WB)], ins[b], isems[b])

        def wait_in(b):
            pltpu.make_async_copy(
                tblT_hbm.at[:, pl.ds(0, _WB)], ins[b], isems[b]).wait()

        def start_out(ki, b):
            pltpu.async_copy(
                ous[b], out_hbm.at[pl.ds(blk(ki), _WB)], osems[b])

        def wait_out(b):
            pltpu.make_async_copy(
                ous[b], out_hbm.at[pl.ds(0, _WB)], osems[b]).wait()

        @pl.when(wid == nw - 1)
        def _tail():
            pltpu.sync_copy(tail_hbm, out_hbm.at[pl.ds(nb * _WB, tail)])

        start_in(0, 0)
        start_in(1, 1)
        lanes = jax.lax.iota(jnp.int32, 16)
        diags = [(lanes + s) % 16 for s in range(16)]

        @pl.loop(0, nk // 2)
        def _body(o):
            for b in range(2):
                ki = o * 2 + b
                wait_in(b)

                @pl.when(ki >= 2)
                def _drain():
                    wait_out(b)

                @pl.loop(0, _WB // 16)
                def _rows(r0):
                    r16 = pl.multiple_of(r0 * 16, 16)
                    row_ids = r16 + lanes
                    for d0 in range(0, D, 16):
                        for s in range(16):
                            feat_ids = d0 + diags[s]
                            v = plsc.load_gather(
                                ins[b], [feat_ids, row_ids])
                            plsc.store_scatter(
                                ous[b], [row_ids, feat_ids], v)

                start_out(ki, b)

                @pl.when(ki + 2 < nk)
                def _refill():
                    start_in(ki + 2, b)

        wait_out(0)
        wait_out(1)

    return tp(tblT, tail128)


def _emb_lookup(idx, table, *, num_cores, num_subcores, chunk):
    B, = idx.shape
    V, D = table.shape
    nw = num_cores * num_subcores
    b_per_w = B // nw
    nchunks = b_per_w // chunk

    mesh = plsc.VectorSubcoreMesh(core_axis_name="c", subcore_axis_name="s")

    @functools.partial(
        pl.kernel,
        out_type=jax.ShapeDtypeStruct((B, D), jnp.float32),
        mesh=mesh,
        compiler_params=pltpu.CompilerParams(
            use_tc_tiling_on_sc=True, skip_device_barrier=True),
        scratch_types=[
            pltpu.VMEM((b_per_w,), jnp.int32),
            pltpu.VMEM((chunk, D), jnp.float32),
            pltpu.VMEM((chunk, D), jnp.float32),
            pltpu.SemaphoreType.DMA,
            pltpu.SemaphoreType.DMA,
            pltpu.SemaphoreType.DMA,
            pltpu.SemaphoreType.DMA,
        ],
    )
    def emb(idx_hbm, table_hbm, out_hbm, idx_v, rows0, rows1,
            gsem0, gsem1, ssem0, ssem1):
        wid = lax.axis_index("s") * num_cores + lax.axis_index("c")
        base = wid * b_per_w
        pltpu.sync_copy(idx_hbm.at[pl.ds(base, b_per_w)], idx_v)

        bufs = (rows0, rows1)
        gsems = (gsem0, gsem1)
        ssems = (ssem0, ssem1)

        def start_gather(g, b):
            pltpu.async_copy(
                table_hbm.at[idx_v.at[pl.ds(g * chunk, chunk)]],
                bufs[b], gsems[b])

        def start_store(g, b):
            pltpu.async_copy(
                bufs[b], out_hbm.at[pl.ds(base + g * chunk, chunk)],
                ssems[b])

        def wait_gather(b):
            pltpu.make_async_copy(
                table_hbm.at[pl.ds(0, chunk)], bufs[b], gsems[b]).wait()

        def wait_store(b):
            pltpu.make_async_copy(
                bufs[b], out_hbm.at[pl.ds(base, chunk)], ssems[b]).wait()

        start_gather(0, 0)
        start_gather(1, 1)

        @pl.loop(0, nchunks // 2)
        def _body(o):
            for b in range(2):
                g = o * 2 + b
                wait_gather(b)
                start_store(g, b)

                @pl.when(g + 2 < nchunks)
                def _refill():
                    wait_store(b)
                    start_gather(g + 2, b)

        wait_store(0)
        wait_store(1)

    return emb(idx, table)


def kernel(X, table):
    B0, B1 = X.shape
    B = B0 * B1
    V, D = table.shape
    idx = X.reshape(B).astype(jnp.int32)
    info = plsc.get_sparse_core_info()
    nw = info.num_cores * info.num_subcores

    nb = V // _WB
    tail128 = jnp.pad(table[nb * _WB:, :], ((0, 0), (0, 128 - D)))
    tbl128 = _transpose_pad(
        table.T, tail128,
        num_cores=info.num_cores, num_subcores=info.num_subcores)

    out = _emb_lookup(
        idx,
        tbl128,
        num_cores=info.num_cores,
        num_subcores=info.num_subcores,
        chunk=256,
    )
    return out.reshape(B0, B1, 128)[:, :, :D]


# speedup vs baseline: 2.1995x; 1.4705x over previous
import functools

import jax
import jax.numpy as jnp
from jax import lax
from jax.experimental import pallas as pl
from jax.experimental.pallas import tpu as pltpu
from jax.experimental.pallas import tpu_sc as plsc

_WB = 256


def _transpose_pad(tblT, tail128, *, num_cores, num_subcores):
    D, V = tblT.shape
    nw = num_cores * num_subcores
    nb = V // _WB
    tail = V - nb * _WB
    nk = (nb + nw - 1) // nw
    nk += nk % 2

    mesh = plsc.VectorSubcoreMesh(core_axis_name="c", subcore_axis_name="s")

    @functools.partial(
        pl.kernel,
        out_type=jax.ShapeDtypeStruct((V, 128), jnp.float32),
        mesh=mesh,
        compiler_params=pltpu.CompilerParams(
            use_tc_tiling_on_sc=True, skip_device_barrier=True,
            needs_layout_passes=False),
        scratch_types=[
            pltpu.VMEM((D, _WB), jnp.float32),
            pltpu.VMEM((D, _WB), jnp.float32),
            pltpu.VMEM((_WB, 128), jnp.float32),
            pltpu.VMEM((_WB, 128), jnp.float32),
            pltpu.SemaphoreType.DMA,
            pltpu.SemaphoreType.DMA,
            pltpu.SemaphoreType.DMA,
            pltpu.SemaphoreType.DMA,
        ],
    )
    def tp(tblT_hbm, tail_hbm, out_hbm, in0, in1, ou0, ou1,
           isem0, isem1, osem0, osem1):
        wid = lax.axis_index("s") * num_cores + lax.axis_index("c")
        ins = (in0, in1)
        ous = (ou0, ou1)
        isems = (isem0, isem1)
        osems = (osem0, osem1)

        def blk(ki):
            k = jnp.minimum(wid + ki * nw, nb - 1)
            return pl.multiple_of(k * _WB, _WB)

        def start_in(ki, b):
            pltpu.async_copy(
                tblT_hbm.at[:, pl.ds(blk(ki), _WB)], ins[b], isems[b])

        def wait_in(b):
            pltpu.make_async_copy(
                tblT_hbm.at[:, pl.ds(0, _WB)], ins[b], isems[b]).wait()

        def start_out(ki, b):
            pltpu.async_copy(
                ous[b], out_hbm.at[pl.ds(blk(ki), _WB)], osems[b])

        def wait_out(b):
            pltpu.make_async_copy(
                ous[b], out_hbm.at[pl.ds(0, _WB)], osems[b]).wait()

        @pl.when(wid == nw - 1)
        def _tail():
            pltpu.sync_copy(tail_hbm, out_hbm.at[pl.ds(nb * _WB, tail)])

        start_in(0, 0)
        start_in(1, 1)
        lanes = jax.lax.iota(jnp.int32, 16)
        diags = [(lanes + s) % 16 for s in range(16)]

        @pl.loop(0, nk // 2)
        def _body(o):
            for b in range(2):
                ki = o * 2 + b
                wait_in(b)

                @pl.when(ki >= 2)
                def _drain():
                    wait_out(b)

                @pl.loop(0, _WB // 16)
                def _rows(r0):
                    r16 = pl.multiple_of(r0 * 16, 16)
                    row_ids = r16 + lanes
                    for d0 in range(0, D, 16):
                        vs = [
                            plsc.load_gather(
                                ins[b], [d0 + diags[s], row_ids])
                            for s in range(16)
                        ]
                        for s in range(16):
                            plsc.store_scatter(
                                ous[b], [row_ids, d0 + diags[s]], vs[s])

                start_out(ki, b)

                @pl.when(ki + 2 < nk)
                def _refill():
                    start_in(ki + 2, b)

        wait_out(0)
        wait_out(1)

    return tp(tblT, tail128)


def _emb_lookup(idx, table, *, num_cores, num_subcores, chunk):
    B, = idx.shape
    V, D = table.shape
    nw = num_cores * num_subcores
    b_per_w = B // nw
    nchunks = b_per_w // chunk

    mesh = plsc.VectorSubcoreMesh(core_axis_name="c", subcore_axis_name="s")

    @functools.partial(
        pl.kernel,
        out_type=jax.ShapeDtypeStruct((B, D), jnp.float32),
        mesh=mesh,
        compiler_params=pltpu.CompilerParams(
            use_tc_tiling_on_sc=True, skip_device_barrier=True),
        scratch_types=[
            pltpu.VMEM((b_per_w,), jnp.int32),
            pltpu.VMEM((chunk, D), jnp.float32),
            pltpu.VMEM((chunk, D), jnp.float32),
            pltpu.SemaphoreType.DMA,
            pltpu.SemaphoreType.DMA,
            pltpu.SemaphoreType.DMA,
            pltpu.SemaphoreType.DMA,
        ],
    )
    def emb(idx_hbm, table_hbm, out_hbm, idx_v, rows0, rows1,
            gsem0, gsem1, ssem0, ssem1):
        wid = lax.axis_index("s") * num_cores + lax.axis_index("c")
        base = wid * b_per_w
        pltpu.sync_copy(idx_hbm.at[pl.ds(base, b_per_w)], idx_v)

        bufs = (rows0, rows1)
        gsems = (gsem0, gsem1)
        ssems = (ssem0, ssem1)

        def start_gather(g, b):
            pltpu.async_copy(
                table_hbm.at[idx_v.at[pl.ds(g * chunk, chunk)]],
                bufs[b], gsems[b])

        def start_store(g, b):
            pltpu.async_copy(
                bufs[b], out_hbm.at[pl.ds(base + g * chunk, chunk)],
                ssems[b])

        def wait_gather(b):
            pltpu.make_async_copy(
                table_hbm.at[pl.ds(0, chunk)], bufs[b], gsems[b]).wait()

        def wait_store(b):
            pltpu.make_async_copy(
                bufs[b], out_hbm.at[pl.ds(base, chunk)], ssems[b]).wait()

        start_gather(0, 0)
        start_gather(1, 1)

        @pl.loop(0, nchunks // 2)
        def _body(o):
            for b in range(2):
                g = o * 2 + b
                wait_gather(b)
                start_store(g, b)

                @pl.when(g + 2 < nchunks)
                def _refill():
                    wait_store(b)
                    start_gather(g + 2, b)

        wait_store(0)
        wait_store(1)

    return emb(idx, table)


def kernel(X, table):
    B0, B1 = X.shape
    B = B0 * B1
    V, D = table.shape
    idx = X.reshape(B).astype(jnp.int32)
    info = plsc.get_sparse_core_info()
    nw = info.num_cores * info.num_subcores

    nb = V // _WB
    tail128 = jnp.pad(table[nb * _WB:, :], ((0, 0), (0, 128 - D)))
    tbl128 = _transpose_pad(
        table.T, tail128,
        num_cores=info.num_cores, num_subcores=info.num_subcores)

    out = _emb_lookup(
        idx,
        tbl128,
        num_cores=info.num_cores,
        num_subcores=info.num_subcores,
        chunk=256,
    )
    return out.reshape(B0, B1, 128)[:, :, :D]


# speedup vs baseline: 2.9213x; 1.3281x over previous
import functools

import jax
import jax.numpy as jnp
from jax import lax
from jax.experimental import pallas as pl
from jax.experimental.pallas import tpu as pltpu
from jax.experimental.pallas import tpu_sc as plsc

_WB = 256


def _transpose_pack(tblT, tailP, *, num_cores, num_subcores):
    D, V = tblT.shape
    nw = num_cores * num_subcores
    nb = V // _WB
    tail_rows = (V - nb * _WB) // 2
    nk = (nb + nw - 1) // nw
    nk += nk % 2
    _OB = _WB // 2

    mesh = plsc.VectorSubcoreMesh(core_axis_name="c", subcore_axis_name="s")

    @functools.partial(
        pl.kernel,
        out_type=jax.ShapeDtypeStruct((V // 2, 2 * D), jnp.float32),
        mesh=mesh,
        compiler_params=pltpu.CompilerParams(
            use_tc_tiling_on_sc=True, skip_device_barrier=True,
            needs_layout_passes=False),
        scratch_types=[
            pltpu.VMEM((D, _WB), jnp.float32),
            pltpu.VMEM((D, _WB), jnp.float32),
            pltpu.VMEM((_OB, 2 * D), jnp.float32),
            pltpu.VMEM((_OB, 2 * D), jnp.float32),
            pltpu.SemaphoreType.DMA,
            pltpu.SemaphoreType.DMA,
            pltpu.SemaphoreType.DMA,
            pltpu.SemaphoreType.DMA,
        ],
    )
    def tp(tblT_hbm, tail_hbm, out_hbm, in0, in1, ou0, ou1,
           isem0, isem1, osem0, osem1):
        wid = lax.axis_index("s") * num_cores + lax.axis_index("c")
        ins = (in0, in1)
        ous = (ou0, ou1)
        isems = (isem0, isem1)
        osems = (osem0, osem1)

        def blk(ki):
            return jnp.minimum(wid + ki * nw, nb - 1)

        def start_in(ki, b):
            off = pl.multiple_of(blk(ki) * _WB, _WB)
            pltpu.async_copy(
                tblT_hbm.at[:, pl.ds(off, _WB)], ins[b], isems[b])

        def wait_in(b):
            pltpu.make_async_copy(
                tblT_hbm.at[:, pl.ds(0, _WB)], ins[b], isems[b]).wait()

        def start_out(ki, b):
            off = pl.multiple_of(blk(ki) * _OB, _OB)
            pltpu.async_copy(
                ous[b], out_hbm.at[pl.ds(off, _OB)], osems[b])

        def wait_out(b):
            pltpu.make_async_copy(
                ous[b], out_hbm.at[pl.ds(0, _OB)], osems[b]).wait()

        @pl.when(wid == nw - 1)
        def _tail():
            pltpu.sync_copy(
                tail_hbm, out_hbm.at[pl.ds(nb * _OB, tail_rows)])

        start_in(0, 0)
        start_in(1, 1)
        lanes = jax.lax.iota(jnp.int32, 16)
        diags = [(lanes + s) % 16 for s in range(16)]
        rowhalf = lanes // 2
        colbase = (lanes % 2) * D

        @pl.loop(0, nk // 2)
        def _body(o):
            for b in range(2):
                ki = o * 2 + b
                wait_in(b)

                @pl.when(ki >= 2)
                def _drain():
                    wait_out(b)

                @pl.loop(0, _WB // 16)
                def _rows(r0):
                    r16 = pl.multiple_of(r0 * 16, 16)
                    row_ids = r16 + lanes
                    prow_ids = r16 // 2 + rowhalf
                    for d0 in range(0, D, 16):
                        vs = [
                            plsc.load_gather(
                                ins[b], [d0 + diags[s], row_ids])
                            for s in range(16)
                        ]
                        for s in range(16):
                            plsc.store_scatter(
                                ous[b],
                                [prow_ids, colbase + d0 + diags[s]],
                                vs[s])

                start_out(ki, b)

                @pl.when(ki + 2 < nk)
                def _refill():
                    start_in(ki + 2, b)

        wait_out(0)
        wait_out(1)

    return tp(tblT, tailP)


def _emb_lookup(idx, table, *, num_cores, num_subcores, chunk):
    B, = idx.shape
    V, D = table.shape
    nw = num_cores * num_subcores
    b_per_w = B // nw
    nchunks = b_per_w // chunk

    mesh = plsc.VectorSubcoreMesh(core_axis_name="c", subcore_axis_name="s")

    @functools.partial(
        pl.kernel,
        out_type=jax.ShapeDtypeStruct((B, 2 * D), jnp.float32),
        mesh=mesh,
        compiler_params=pltpu.CompilerParams(
            use_tc_tiling_on_sc=False, skip_device_barrier=True),
        scratch_types=[
            pltpu.VMEM((b_per_w,), jnp.int32),
            pltpu.VMEM((chunk, D), jnp.float32),
            pltpu.VMEM((chunk, D), jnp.float32),
            pltpu.SemaphoreType.DMA,
            pltpu.SemaphoreType.DMA,
            pltpu.SemaphoreType.DMA,
            pltpu.SemaphoreType.DMA,
        ],
    )
    def emb(idx_hbm, table_hbm, out_hbm, idx_v, rows0, rows1,
            gsem0, gsem1, ssem0, ssem1):
        wid = lax.axis_index("s") * num_cores + lax.axis_index("c")
        base = wid * b_per_w
        base2 = wid * (b_per_w // 2)
        pltpu.sync_copy(idx_hbm.at[pl.ds(base, b_per_w)], idx_v)

        bufs = (rows0, rows1)
        gsems = (gsem0, gsem1)
        ssems = (ssem0, ssem1)
        c2 = chunk // 2

        def start_gather(g, b):
            pltpu.async_copy(
                table_hbm.at[idx_v.at[pl.ds(g * chunk, chunk)]],
                bufs[b], gsems[b])

        def start_store(g, b):
            pltpu.async_copy(
                bufs[b],
                out_hbm.at[pl.ds(base + g * chunk, chunk), pl.ds(0, D)],
                ssems[b])

        def wait_gather(b):
            pltpu.make_async_copy(
                table_hbm.at[pl.ds(0, chunk)], bufs[b], gsems[b]).wait()

        def wait_store(b):
            pltpu.make_async_copy(
                bufs[b],
                out_hbm.at[pl.ds(base, chunk), pl.ds(0, D)],
                ssems[b]).wait()

        start_gather(0, 0)
        start_gather(1, 1)

        @pl.loop(0, nchunks // 2)
        def _body(o):
            for b in range(2):
                g = o * 2 + b
                wait_gather(b)
                start_store(g, b)

                @pl.when(g + 2 < nchunks)
                def _refill():
                    wait_store(b)
                    start_gather(g + 2, b)

        wait_store(0)
        wait_store(1)

    return emb(idx, table)


def kernel(X, table):
    B0, B1 = X.shape
    B = B0 * B1
    V, D = table.shape
    idx = X.reshape(B).astype(jnp.int32)
    info = plsc.get_sparse_core_info()

    nb = V // _WB
    tailP = table[nb * _WB:, :].reshape(-1, 2 * D)
    tblP = _transpose_pack(
        table.T, tailP,
        num_cores=info.num_cores, num_subcores=info.num_subcores)

    out = _emb_lookup(
        idx,
        tblP.reshape(V, D),
        num_cores=info.num_cores,
        num_subcores=info.num_subcores,
        chunk=512,
    )
    return out.reshape(B0, B1, 2 * D)[:, :, :D]


# speedup vs baseline: 3.0041x; 1.0283x over previous
import functools

import jax
import jax.numpy as jnp
from jax import lax
from jax.experimental import pallas as pl
from jax.experimental.pallas import tpu as pltpu
from jax.experimental.pallas import tpu_sc as plsc

_WB = 384


def _transpose_pack(tblT, tailP, *, num_cores, num_subcores):
    D, V = tblT.shape
    nw = num_cores * num_subcores
    nb = V // _WB
    tail_rows = (V - nb * _WB) // 2
    nk = (nb + nw - 1) // nw
    nk += nk % 2
    _OB = _WB // 2

    mesh = plsc.VectorSubcoreMesh(core_axis_name="c", subcore_axis_name="s")

    @functools.partial(
        pl.kernel,
        out_type=jax.ShapeDtypeStruct((V // 2, 2 * D), jnp.float32),
        mesh=mesh,
        compiler_params=pltpu.CompilerParams(
            use_tc_tiling_on_sc=True, skip_device_barrier=True,
            needs_layout_passes=False),
        scratch_types=[
            pltpu.VMEM((D, _WB), jnp.float32),
            pltpu.VMEM((D, _WB), jnp.float32),
            pltpu.VMEM((_OB, 2 * D), jnp.float32),
            pltpu.VMEM((_OB, 2 * D), jnp.float32),
            pltpu.SemaphoreType.DMA,
            pltpu.SemaphoreType.DMA,
            pltpu.SemaphoreType.DMA,
            pltpu.SemaphoreType.DMA,
        ],
    )
    def tp(tblT_hbm, tail_hbm, out_hbm, in0, in1, ou0, ou1,
           isem0, isem1, osem0, osem1):
        wid = lax.axis_index("s") * num_cores + lax.axis_index("c")
        ins = (in0, in1)
        ous = (ou0, ou1)
        isems = (isem0, isem1)
        osems = (osem0, osem1)

        def blk(ki):
            return jnp.minimum(wid + ki * nw, nb - 1)

        def start_in(ki, b):
            off = pl.multiple_of(blk(ki) * _WB, _WB)
            pltpu.async_copy(
                tblT_hbm.at[:, pl.ds(off, _WB)], ins[b], isems[b])

        def wait_in(b):
            pltpu.make_async_copy(
                tblT_hbm.at[:, pl.ds(0, _WB)], ins[b], isems[b]).wait()

        def start_out(ki, b):
            off = pl.multiple_of(blk(ki) * _OB, _OB)
            pltpu.async_copy(
                ous[b], out_hbm.at[pl.ds(off, _OB)], osems[b])

        def wait_out(b):
            pltpu.make_async_copy(
                ous[b], out_hbm.at[pl.ds(0, _OB)], osems[b]).wait()

        @pl.when(wid == nw - 1)
        def _tail():
            pltpu.sync_copy(
                tail_hbm, out_hbm.at[pl.ds(nb * _OB, tail_rows)])

        start_in(0, 0)
        start_in(1, 1)
        lanes = jax.lax.iota(jnp.int32, 16)
        diags = [(lanes + s) % 16 for s in range(16)]
        rowhalf = lanes // 2
        colbase = (lanes % 2) * D

        @pl.loop(0, nk // 2)
        def _body(o):
            for b in range(2):
                ki = o * 2 + b
                wait_in(b)

                @pl.when(ki >= 2)
                def _drain():
                    wait_out(b)

                @pl.loop(0, _WB // 16)
                def _rows(r0):
                    r16 = pl.multiple_of(r0 * 16, 16)
                    row_ids = r16 + lanes
                    prow_ids = r16 // 2 + rowhalf
                    for d0 in range(0, D, 16):
                        vs = [
                            plsc.load_gather(
                                ins[b], [d0 + diags[s], row_ids])
                            for s in range(16)
                        ]
                        for s in range(16):
                            plsc.store_scatter(
                                ous[b],
                                [prow_ids, colbase + d0 + diags[s]],
                                vs[s])

                start_out(ki, b)

                @pl.when(ki + 2 < nk)
                def _refill():
                    start_in(ki + 2, b)

        wait_out(0)
        wait_out(1)

    return tp(tblT, tailP)


def _emb_lookup(idx, table, *, num_cores, num_subcores, chunk):
    B, = idx.shape
    V, D = table.shape
    nw = num_cores * num_subcores
    b_per_w = B // nw
    nchunks = b_per_w // chunk

    mesh = plsc.VectorSubcoreMesh(core_axis_name="c", subcore_axis_name="s")

    @functools.partial(
        pl.kernel,
        out_type=jax.ShapeDtypeStruct((B, 2 * D), jnp.float32),
        mesh=mesh,
        compiler_params=pltpu.CompilerParams(
            use_tc_tiling_on_sc=False, skip_device_barrier=True),
        scratch_types=[
            pltpu.VMEM((b_per_w,), jnp.int32),
            pltpu.VMEM((chunk, D), jnp.float32),
            pltpu.VMEM((chunk, D), jnp.float32),
            pltpu.SemaphoreType.DMA,
            pltpu.SemaphoreType.DMA,
            pltpu.SemaphoreType.DMA,
            pltpu.SemaphoreType.DMA,
        ],
    )
    def emb(idx_hbm, table_hbm, out_hbm, idx_v, rows0, rows1,
            gsem0, gsem1, ssem0, ssem1):
        wid = lax.axis_index("s") * num_cores + lax.axis_index("c")
        base = wid * b_per_w
        base2 = wid * (b_per_w // 2)
        pltpu.sync_copy(idx_hbm.at[pl.ds(base, b_per_w)], idx_v)

        bufs = (rows0, rows1)
        gsems = (gsem0, gsem1)
        ssems = (ssem0, ssem1)
        c2 = chunk // 2

        def start_gather(g, b):
            pltpu.async_copy(
                table_hbm.at[idx_v.at[pl.ds(g * chunk, chunk)]],
                bufs[b], gsems[b])

        def start_store(g, b):
            pltpu.async_copy(
                bufs[b],
                out_hbm.at[pl.ds(base + g * chunk, chunk), pl.ds(0, D)],
                ssems[b])

        def wait_gather(b):
            pltpu.make_async_copy(
                table_hbm.at[pl.ds(0, chunk)], bufs[b], gsems[b]).wait()

        def wait_store(b):
            pltpu.make_async_copy(
                bufs[b],
                out_hbm.at[pl.ds(base, chunk), pl.ds(0, D)],
                ssems[b]).wait()

        start_gather(0, 0)
        start_gather(1, 1)

        @pl.loop(0, nchunks // 2)
        def _body(o):
            for b in range(2):
                g = o * 2 + b
                wait_gather(b)
                start_store(g, b)

                @pl.when(g + 2 < nchunks)
                def _refill():
                    wait_store(b)
                    start_gather(g + 2, b)

        wait_store(0)
        wait_store(1)

    return emb(idx, table)


def kernel(X, table):
    B0, B1 = X.shape
    B = B0 * B1
    V, D = table.shape
    idx = X.reshape(B).astype(jnp.int32)
    info = plsc.get_sparse_core_info()

    nb = V // _WB
    tailP = table[nb * _WB:, :].reshape(-1, 2 * D)
    tblP = _transpose_pack(
        table.T, tailP,
        num_cores=info.num_cores, num_subcores=info.num_subcores)

    out = _emb_lookup(
        idx,
        tblP.reshape(V, D),
        num_cores=info.num_cores,
        num_subcores=info.num_subcores,
        chunk=512,
    )
    return out.reshape(B0, B1, 2 * D)[:, :, :D]
